# baseline (speedup 1.0000x reference)
import functools

import jax
import jax.numpy as jnp
from jax import lax
from jax.experimental import pallas as pl
from jax.experimental.pallas import tpu as pltpu
from jax.experimental.pallas import tpu_sc as plsc

NUM_E = 1024
DIM = 64
BATCH = 16384
BETA = 0.25

HALVES = 2
HALF = BATCH // HALVES
TILE = 1024
GRID = HALF // TILE

SC_CORES = 2
SC_SUBCORES = 16
SC_WORKERS = SC_CORES * SC_SUBCORES
ROWS_PER_WORKER = HALF // SC_WORKERS


def _vq_argmin_kernel(z_ref, w_ref, idx_ref, msum_ref):
    i = pl.program_id(0)
    z = z_ref[...]
    w = w_ref[...]
    znorm = jnp.sum(z * z, axis=1, keepdims=True)
    wnorm = jnp.sum(w * w, axis=1)
    t = 2.0 * lax.dot_general(z, w, (((1,), (1,)), ((), ())))
    d = (znorm + wnorm[None, :]) - t
    m = jnp.min(d, axis=1)
    idx_ref[0, 0, :] = jnp.argmin(d, axis=1).astype(jnp.int32)

    @pl.when(i == 0)
    def _init():
        msum_ref[0, 0] = 0.0

    msum_ref[0, 0] += jnp.sum(m)


def _tc_argmin(z, w, half):
    off = half * GRID
    return pl.pallas_call(
        _vq_argmin_kernel,
        grid=(GRID,),
        in_specs=[
            pl.BlockSpec((TILE, DIM), lambda i: (i + off, 0)),
            pl.BlockSpec((NUM_E, DIM), lambda i: (0, 0)),
        ],
        out_specs=[
            pl.BlockSpec((1, 1, TILE), lambda i: (i, 0, 0)),
            pl.BlockSpec(memory_space=pltpu.SMEM),
        ],
        out_shape=[
            jax.ShapeDtypeStruct((GRID, 1, TILE), jnp.int32),
            jax.ShapeDtypeStruct((1, 1), jnp.float32),
        ],
        compiler_params=pltpu.CompilerParams(
            dimension_semantics=("arbitrary",),
        ),
    )(z, w)


@functools.cache
def _make_sc_gather():
    @functools.partial(
        pl.kernel,
        mesh=plsc.VectorSubcoreMesh(core_axis_name="c", subcore_axis_name="s"),
        out_type=jax.ShapeDtypeStruct((HALF, DIM), jnp.float32),
        scratch_types=[
            pltpu.VMEM((ROWS_PER_WORKER,), jnp.int32),
            pltpu.VMEM((ROWS_PER_WORKER, DIM), jnp.float32),
            pltpu.SemaphoreType.DMA,
        ],
        compiler_params=pltpu.CompilerParams(use_tc_tiling_on_sc=False),
    )
    def _sc_gather(table_hbm, idx_hbm, out_hbm, idx_v, rows_v, sem):
        wid = lax.axis_index("s") * SC_CORES + lax.axis_index("c")
        base = wid * ROWS_PER_WORKER
        pltpu.sync_copy(idx_hbm.at[pl.ds(base, ROWS_PER_WORKER)], idx_v)
        pltpu.async_copy(table_hbm.at[idx_v], rows_v, sem).wait()
        pltpu.sync_copy(rows_v, out_hbm.at[pl.ds(base, ROWS_PER_WORKER)])

    return _sc_gather


def kernel(z, embedding_weight):
    sc_gather = _make_sc_gather()
    zq_halves = []
    msums = []
    for h in range(HALVES):
        idx3, msum = _tc_argmin(z, embedding_weight, h)
        zq_halves.append(sc_gather(embedding_weight, idx3.reshape(HALF)))
        msums.append(msum[0, 0])
    z_q = jnp.concatenate(zq_halves, axis=0)
    vq_loss = (msums[0] + msums[1]) * ((1.0 + BETA) / (BATCH * DIM))
    return (z_q, jnp.reshape(vq_loss, ()))


# speedup vs baseline: 1.0220x; 1.0220x over previous
import functools

import jax
import jax.numpy as jnp
from jax import lax
from jax.experimental import pallas as pl
from jax.experimental.pallas import tpu as pltpu
from jax.experimental.pallas import tpu_sc as plsc

NUM_E = 1024
DIM = 64
BATCH = 16384
BETA = 0.25

HALVES = 2
HALF = BATCH // HALVES
TILE = 1024
GRID = HALF // TILE

SC_CORES = 2
SC_SUBCORES = 16
SC_WORKERS = SC_CORES * SC_SUBCORES
ROWS_PER_WORKER = HALF // SC_WORKERS


def _vq_argmin_kernel(z_ref, w_ref, idx_ref, msum_ref):
    i = pl.program_id(0)
    z = z_ref[...]
    w = w_ref[...]
    znorm = jnp.sum(z * z, axis=1, keepdims=True)
    wnorm = jnp.sum(w * w, axis=1)
    t = 2.0 * lax.dot_general(z, w, (((1,), (1,)), ((), ())))
    d = (znorm + wnorm[None, :]) - t
    m = jnp.min(d, axis=1, keepdims=True)
    col = lax.broadcasted_iota(jnp.int32, d.shape, 1)
    idx = jnp.min(jnp.where(d == m, col, NUM_E), axis=1)
    idx_ref[0, 0, :] = idx

    @pl.when(i == 0)
    def _init():
        msum_ref[0, 0] = 0.0

    msum_ref[0, 0] += jnp.sum(m)


def _tc_argmin(z, w, half):
    off = half * GRID
    return pl.pallas_call(
        _vq_argmin_kernel,
        grid=(GRID,),
        in_specs=[
            pl.BlockSpec((TILE, DIM), lambda i: (i + off, 0)),
            pl.BlockSpec((NUM_E, DIM), lambda i: (0, 0)),
        ],
        out_specs=[
            pl.BlockSpec((1, 1, TILE), lambda i: (i, 0, 0)),
            pl.BlockSpec(memory_space=pltpu.SMEM),
        ],
        out_shape=[
            jax.ShapeDtypeStruct((GRID, 1, TILE), jnp.int32),
            jax.ShapeDtypeStruct((1, 1), jnp.float32),
        ],
        compiler_params=pltpu.CompilerParams(
            dimension_semantics=("arbitrary",),
        ),
    )(z, w)


@functools.cache
def _make_sc_gather():
    @functools.partial(
        pl.kernel,
        mesh=plsc.VectorSubcoreMesh(core_axis_name="c", subcore_axis_name="s"),
        out_type=jax.ShapeDtypeStruct((HALF, DIM), jnp.float32),
        scratch_types=[
            pltpu.VMEM((ROWS_PER_WORKER,), jnp.int32),
            pltpu.VMEM((ROWS_PER_WORKER, DIM), jnp.float32),
            pltpu.SemaphoreType.DMA,
        ],
        compiler_params=pltpu.CompilerParams(use_tc_tiling_on_sc=False),
    )
    def _sc_gather(table_hbm, idx_hbm, out_hbm, idx_v, rows_v, sem):
        wid = lax.axis_index("s") * SC_CORES + lax.axis_index("c")
        base = wid * ROWS_PER_WORKER
        pltpu.sync_copy(idx_hbm.at[pl.ds(base, ROWS_PER_WORKER)], idx_v)
        pltpu.async_copy(table_hbm.at[idx_v], rows_v, sem).wait()
        pltpu.sync_copy(rows_v, out_hbm.at[pl.ds(base, ROWS_PER_WORKER)])

    return _sc_gather


def kernel(z, embedding_weight):
    sc_gather = _make_sc_gather()
    zq_halves = []
    msums = []
    for h in range(HALVES):
        idx3, msum = _tc_argmin(z, embedding_weight, h)
        zq_halves.append(sc_gather(embedding_weight, idx3.reshape(HALF)))
        msums.append(msum[0, 0])
    z_q = jnp.concatenate(zq_halves, axis=0)
    vq_loss = (msums[0] + msums[1]) * ((1.0 + BETA) / (BATCH * DIM))
    return (z_q, jnp.reshape(vq_loss, ()))


# speedup vs baseline: 1.0706x; 1.0476x over previous
import functools

import jax
import jax.numpy as jnp
from jax import lax
from jax.experimental import pallas as pl
from jax.experimental.pallas import tpu as pltpu
from jax.experimental.pallas import tpu_sc as plsc

NUM_E = 1024
DIM = 64
BATCH = 16384
BETA = 0.25

TILE = 1024
GRID = BATCH // TILE
LANES = 128
NCHUNK = NUM_E // LANES

SC_CORES = 2
SC_SUBCORES = 16
SC_WORKERS = SC_CORES * SC_SUBCORES
ROWS_PER_WORKER = BATCH // SC_WORKERS
IDX_PER_ROW = TILE // ROWS_PER_WORKER


def _vq_argmin_kernel(z_ref, w_ref, idx_ref, msum_ref):
    i = pl.program_id(0)
    z = z_ref[...]
    w = w_ref[...]
    znorm = jnp.sum(z * z, axis=1, keepdims=True)
    wnorm = jnp.sum(w * w, axis=1)
    t = 2.0 * lax.dot_general(z, w, (((1,), (1,)), ((), ())))

    mval = None
    mchunk = None
    for k in range(NCHUNK):
        tk = lax.slice_in_dim(t, k * LANES, (k + 1) * LANES, axis=1)
        wk = lax.slice_in_dim(wnorm, k * LANES, (k + 1) * LANES, axis=0)
        dk = (znorm + wk[None, :]) - tk
        if k == 0:
            mval = dk
            mchunk = jnp.zeros(dk.shape, jnp.int32)
        else:
            lt = dk < mval
            mval = jnp.where(lt, dk, mval)
            mchunk = jnp.where(lt, k, mchunk)

    m = jnp.min(mval, axis=1, keepdims=True)
    lane = lax.broadcasted_iota(jnp.int32, mval.shape, 1)
    fullidx = mchunk * LANES + lane
    idx = jnp.min(jnp.where(mval == m, fullidx, NUM_E), axis=1)
    idx_ref[0, 0, :] = idx

    @pl.when(i == 0)
    def _init():
        msum_ref[0, 0] = 0.0

    msum_ref[0, 0] += jnp.sum(m)


def _tc_argmin(z, w):
    return pl.pallas_call(
        _vq_argmin_kernel,
        grid=(GRID,),
        in_specs=[
            pl.BlockSpec((TILE, DIM), lambda i: (i, 0)),
            pl.BlockSpec((NUM_E, DIM), lambda i: (0, 0)),
        ],
        out_specs=[
            pl.BlockSpec((1, 1, TILE), lambda i: (i, 0, 0)),
            pl.BlockSpec(memory_space=pltpu.SMEM),
        ],
        out_shape=[
            jax.ShapeDtypeStruct((GRID, 1, TILE), jnp.int32),
            jax.ShapeDtypeStruct((1, 1), jnp.float32),
        ],
        compiler_params=pltpu.CompilerParams(
            dimension_semantics=("arbitrary",),
        ),
    )(z, w)


@functools.cache
def _make_sc_gather():
    @functools.partial(
        pl.kernel,
        mesh=plsc.VectorSubcoreMesh(core_axis_name="c", subcore_axis_name="s"),
        out_type=jax.ShapeDtypeStruct((BATCH, DIM), jnp.float32),
        scratch_types=[
            pltpu.VMEM((ROWS_PER_WORKER,), jnp.int32),
            pltpu.VMEM((ROWS_PER_WORKER, DIM), jnp.float32),
            pltpu.SemaphoreType.DMA,
        ],
        compiler_params=pltpu.CompilerParams(use_tc_tiling_on_sc=False),
    )
    def _sc_gather(table_hbm, idx_hbm, out_hbm, idx_v, rows_v, sem):
        wid = lax.axis_index("s") * SC_CORES + lax.axis_index("c")
        base = wid * ROWS_PER_WORKER
        row = wid // IDX_PER_ROW
        off = (wid % IDX_PER_ROW) * ROWS_PER_WORKER
        pltpu.sync_copy(idx_hbm.at[row, 0, pl.ds(off, ROWS_PER_WORKER)], idx_v)
        pltpu.async_copy(table_hbm.at[idx_v], rows_v, sem).wait()
        pltpu.sync_copy(rows_v, out_hbm.at[pl.ds(base, ROWS_PER_WORKER)])

    return _sc_gather


def kernel(z, embedding_weight):
    idx3, msum = _tc_argmin(z, embedding_weight)
    z_q = _make_sc_gather()(embedding_weight, idx3)
    vq_loss = jnp.reshape(msum[0, 0] * ((1.0 + BETA) / (BATCH * DIM)), ())
    return (z_q, vq_loss)


# speedup vs baseline: 1.3347x; 1.2467x over previous
import functools

import jax
import jax.numpy as jnp
from jax import lax
from jax.experimental import pallas as pl
from jax.experimental.pallas import tpu as pltpu
from jax.experimental.pallas import tpu_sc as plsc

NUM_E = 1024
DIM = 64
BATCH = 16384
BETA = 0.25

TILE = 1024
GRID = BATCH // TILE
LANES = 128
NCHUNK = NUM_E // LANES
IDX_ROWS = BATCH // LANES

SC_CORES = 2
SC_SUBCORES = 16
SC_WORKERS = SC_CORES * SC_SUBCORES
IDXROWS_PER_WORKER = IDX_ROWS // SC_WORKERS


def _vq_argmin_kernel(z_ref, w_ref, idx_ref, msum_ref):
    i = pl.program_id(0)
    z = z_ref[...]
    w = w_ref[...]
    znorm = jnp.sum(z * z, axis=1, keepdims=True)
    wnorm = jnp.sum(w * w, axis=1)
    t = 2.0 * lax.dot_general(z, w, (((1,), (1,)), ((), ())))

    mval = None
    mchunk = None
    for k in range(NCHUNK):
        tk = lax.slice_in_dim(t, k * LANES, (k + 1) * LANES, axis=1)
        wk = lax.slice_in_dim(wnorm, k * LANES, (k + 1) * LANES, axis=0)
        dk = (znorm + wk[None, :]) - tk
        if k == 0:
            mval = dk
            mchunk = jnp.zeros(dk.shape, jnp.int32)
        else:
            lt = dk < mval
            mval = jnp.where(lt, dk, mval)
            mchunk = jnp.where(lt, k, mchunk)

    m = jnp.min(mval, axis=1, keepdims=True)
    lane = lax.broadcasted_iota(jnp.int32, mval.shape, 1)
    fullidx = mchunk * LANES + lane
    idx = jnp.min(jnp.where(mval == m, fullidx, NUM_E), axis=1)
    idx_ref[...] = idx.reshape(TILE // LANES, LANES)

    @pl.when(i == 0)
    def _init():
        msum_ref[0, 0] = 0.0

    msum_ref[0, 0] += jnp.sum(m)


def _tc_argmin(z, w):
    return pl.pallas_call(
        _vq_argmin_kernel,
        grid=(GRID,),
        in_specs=[
            pl.BlockSpec((TILE, DIM), lambda i: (i, 0)),
            pl.BlockSpec((NUM_E, DIM), lambda i: (0, 0)),
        ],
        out_specs=[
            pl.BlockSpec((TILE // LANES, LANES), lambda i: (i, 0)),
            pl.BlockSpec(memory_space=pltpu.SMEM),
        ],
        out_shape=[
            jax.ShapeDtypeStruct((IDX_ROWS, LANES), jnp.int32),
            jax.ShapeDtypeStruct((1, 1), jnp.float32),
        ],
        compiler_params=pltpu.CompilerParams(
            dimension_semantics=("arbitrary",),
        ),
    )(z, w)


@functools.cache
def _make_sc_gather():
    @functools.partial(
        pl.kernel,
        mesh=plsc.VectorSubcoreMesh(core_axis_name="c", subcore_axis_name="s"),
        out_type=jax.ShapeDtypeStruct((BATCH, LANES), jnp.float32),
        scratch_types=[
            pltpu.VMEM((LANES,), jnp.int32),
            pltpu.VMEM((LANES, LANES), jnp.float32),
            pltpu.SemaphoreType.DMA,
        ],
    )
    def _sc_gather(table_hbm, idx_hbm, out_hbm, idx_v, rows_v, sem):
        wid = lax.axis_index("s") * SC_CORES + lax.axis_index("c")
        for r in range(IDXROWS_PER_WORKER):
            row = wid * IDXROWS_PER_WORKER + r
            pltpu.sync_copy(idx_hbm.at[row], idx_v)
            pltpu.async_copy(table_hbm.at[idx_v], rows_v, sem).wait()
            pltpu.sync_copy(rows_v, out_hbm.at[pl.ds(row * LANES, LANES)])

    return _sc_gather


def kernel(z, embedding_weight):
    idx2, msum = _tc_argmin(z, embedding_weight)
    w128 = jnp.pad(embedding_weight, ((0, 0), (0, LANES - DIM)))
    z_q = _make_sc_gather()(w128, idx2)[:, :DIM]
    vq_loss = jnp.reshape(msum[0, 0] * ((1.0 + BETA) / (BATCH * DIM)), ())
    return (z_q, vq_loss)


# speedup vs baseline: 1.5548x; 1.1649x over previous
import functools

import jax
import jax.numpy as jnp
from jax import lax
from jax.experimental import pallas as pl
from jax.experimental.pallas import tpu as pltpu
from jax.experimental.pallas import tpu_sc as plsc

NUM_E = 1024
DIM = 64
BATCH = 16384
BETA = 0.25

TILE = 1024
GRID = BATCH // TILE
LANES = 128
NCHUNK = NUM_E // LANES
IDX_ROWS = BATCH // LANES

SC_CORES = 2
SC_SUBCORES = 16
SC_WORKERS = SC_CORES * SC_SUBCORES
IDXROWS_PER_WORKER = IDX_ROWS // SC_WORKERS


def _vq_argmin_kernel(zT_ref, wT_ref, idx_ref, msum_ref):
    i = pl.program_id(0)
    zT = zT_ref[...]
    wT = wT_ref[...]
    znorm = jnp.sum(zT * zT, axis=0, keepdims=True)
    wnorm = jnp.sum(wT * wT, axis=0, keepdims=True)
    wnormc = jnp.swapaxes(wnorm, 0, 1)
    tT = 2.0 * lax.dot_general(wT, zT, (((0,), (0,)), ((), ())))

    mval = None
    mchunk = None
    for k in range(NCHUNK):
        tk = lax.slice_in_dim(tT, k * LANES, (k + 1) * LANES, axis=0)
        wk = lax.slice_in_dim(wnormc, k * LANES, (k + 1) * LANES, axis=0)
        dk = (znorm + wk) - tk
        if k == 0:
            mval = dk
            mchunk = jnp.zeros(dk.shape, jnp.int32)
        else:
            lt = dk < mval
            mval = jnp.where(lt, dk, mval)
            mchunk = jnp.where(lt, k, mchunk)

    m = jnp.min(mval, axis=0, keepdims=True)
    row = lax.broadcasted_iota(jnp.int32, mval.shape, 0)
    fullidx = mchunk * LANES + row
    idx = jnp.min(jnp.where(mval == m, fullidx, NUM_E), axis=0)
    idx_ref[...] = idx.reshape(TILE // LANES, LANES)

    @pl.when(i == 0)
    def _init():
        msum_ref[0, 0] = 0.0

    msum_ref[0, 0] += jnp.sum(m)


def _tc_argmin(zT, wT):
    return pl.pallas_call(
        _vq_argmin_kernel,
        grid=(GRID,),
        in_specs=[
            pl.BlockSpec((DIM, TILE), lambda i: (0, i)),
            pl.BlockSpec((DIM, NUM_E), lambda i: (0, 0)),
        ],
        out_specs=[
            pl.BlockSpec((TILE // LANES, LANES), lambda i: (i, 0)),
            pl.BlockSpec(memory_space=pltpu.SMEM),
        ],
        out_shape=[
            jax.ShapeDtypeStruct((IDX_ROWS, LANES), jnp.int32),
            jax.ShapeDtypeStruct((1, 1), jnp.float32),
        ],
        compiler_params=pltpu.CompilerParams(
            dimension_semantics=("arbitrary",),
        ),
    )(zT, wT)


DIMS_PER_WORKER = DIM // SC_WORKERS
SC_VLEN = 16


@functools.cache
def _make_sc_gather():
    @functools.partial(
        pl.kernel,
        mesh=plsc.VectorSubcoreMesh(core_axis_name="c", subcore_axis_name="s"),
        out_type=jax.ShapeDtypeStruct((DIM, BATCH), jnp.float32),
        scratch_types=[
            pltpu.VMEM((DIMS_PER_WORKER * NUM_E,), jnp.float32),
            pltpu.VMEM((IDX_ROWS, LANES), jnp.int32),
            pltpu.VMEM((DIMS_PER_WORKER, BATCH), jnp.float32),
        ],
        compiler_params=pltpu.CompilerParams(needs_layout_passes=False),
    )
    def _sc_gather(wt_hbm, idx_hbm, out_hbm, wt_v, idx_v, out_v):
        wid = lax.axis_index("s") * SC_CORES + lax.axis_index("c")
        d0 = wid * DIMS_PER_WORKER
        for d in range(DIMS_PER_WORKER):
            pltpu.sync_copy(wt_hbm.at[d0 + d],
                            wt_v.at[pl.ds(d * NUM_E, NUM_E)])
        pltpu.sync_copy(idx_hbm, idx_v)

        def body(r, _):
            for j in range(LANES // SC_VLEN):
                idx16 = idx_v[r, pl.ds(j * SC_VLEN, SC_VLEN)]
                for d in range(DIMS_PER_WORKER):
                    vals = plsc.load_gather(wt_v, [idx16 + (d * NUM_E)])
                    out_v[d, pl.ds(r * LANES + j * SC_VLEN, SC_VLEN)] = vals
            return _

        lax.fori_loop(0, IDX_ROWS, body, None)
        pltpu.sync_copy(out_v, out_hbm.at[pl.ds(d0, DIMS_PER_WORKER)])

    return _sc_gather


def kernel(z, embedding_weight):
    idx2, msum = _tc_argmin(z.T, embedding_weight.T)
    z_q = _make_sc_gather()(embedding_weight.T, idx2).T
    vq_loss = jnp.reshape(msum[0, 0] * ((1.0 + BETA) / (BATCH * DIM)), ())
    return (z_q, vq_loss)


# speedup vs baseline: 1.8554x; 1.1933x over previous
import functools

import jax
import jax.numpy as jnp
from jax import lax
from jax.experimental import pallas as pl
from jax.experimental.pallas import tpu as pltpu
from jax.experimental.pallas import tpu_sc as plsc

NUM_E = 1024
DIM = 64
BATCH = 16384
BETA = 0.25

TILE = 1024
GRID = BATCH // TILE
LANES = 128
NCHUNK = NUM_E // LANES
IDX_ROWS = BATCH // LANES

SC_CORES = 2
SC_SUBCORES = 16
SC_WORKERS = SC_CORES * SC_SUBCORES
IDXROWS_PER_WORKER = IDX_ROWS // SC_WORKERS


def _vq_argmin_kernel(zT_ref, wT_ref, idx_ref, msum_ref):
    i = pl.program_id(0)
    zT = zT_ref[...]
    wT = wT_ref[...]
    znorm = jnp.sum(zT * zT, axis=0, keepdims=True)
    wnorm = jnp.sum(wT * wT, axis=0, keepdims=True)
    wnormc = jnp.swapaxes(wnorm, 0, 1)
    tT = 2.0 * lax.dot_general(wT, zT, (((0,), (0,)), ((), ())))

    mval = None
    mchunk = None
    for k in range(NCHUNK):
        tk = lax.slice_in_dim(tT, k * LANES, (k + 1) * LANES, axis=0)
        wk = lax.slice_in_dim(wnormc, k * LANES, (k + 1) * LANES, axis=0)
        dk = (znorm + wk) - tk
        if k == 0:
            mval = dk
            mchunk = jnp.zeros(dk.shape, jnp.int32)
        else:
            lt = dk < mval
            mval = jnp.where(lt, dk, mval)
            mchunk = jnp.where(lt, k, mchunk)

    m = jnp.min(mval, axis=0, keepdims=True)
    row = lax.broadcasted_iota(jnp.int32, mval.shape, 0)
    fullidx = mchunk * LANES + row
    idx = jnp.min(jnp.where(mval == m, fullidx, NUM_E), axis=0)
    idx_ref[...] = idx.reshape(TILE // LANES, LANES)

    @pl.when(i == 0)
    def _init():
        msum_ref[0, 0] = 0.0

    msum_ref[0, 0] += jnp.sum(m)


def _tc_argmin(zT, wT):
    return pl.pallas_call(
        _vq_argmin_kernel,
        grid=(GRID,),
        in_specs=[
            pl.BlockSpec((DIM, TILE), lambda i: (0, i)),
            pl.BlockSpec((DIM, NUM_E), lambda i: (0, 0)),
        ],
        out_specs=[
            pl.BlockSpec((TILE // LANES, LANES), lambda i: (i, 0)),
            pl.BlockSpec(memory_space=pltpu.SMEM),
        ],
        out_shape=[
            jax.ShapeDtypeStruct((IDX_ROWS, LANES), jnp.int32),
            jax.ShapeDtypeStruct((1, 1), jnp.float32),
        ],
        compiler_params=pltpu.CompilerParams(
            dimension_semantics=("arbitrary",),
        ),
    )(zT, wT)


DIMS_PER_WORKER = DIM // SC_WORKERS
SC_VLEN = 16


@functools.cache
def _make_sc_gather():
    @functools.partial(
        pl.kernel,
        mesh=plsc.VectorSubcoreMesh(core_axis_name="c", subcore_axis_name="s"),
        out_type=jax.ShapeDtypeStruct((DIM, BATCH), jnp.float32),
        scratch_types=[
            pltpu.VMEM((DIMS_PER_WORKER * NUM_E,), jnp.float32),
            pltpu.VMEM((IDX_ROWS, LANES), jnp.int32),
            pltpu.VMEM((DIMS_PER_WORKER, BATCH), jnp.float32),
        ],
        compiler_params=pltpu.CompilerParams(needs_layout_passes=False),
    )
    def _sc_gather(wt_hbm, idx_hbm, out_hbm, wt_v, idx_v, out_v):
        wid = lax.axis_index("s") * SC_CORES + lax.axis_index("c")
        d0 = wid * DIMS_PER_WORKER
        for d in range(DIMS_PER_WORKER):
            pltpu.sync_copy(wt_hbm.at[d0 + d],
                            wt_v.at[pl.ds(d * NUM_E, NUM_E)])
        pltpu.sync_copy(idx_hbm, idx_v)

        @plsc.parallel_loop(0, IDX_ROWS, unroll=2)
        def _body(r):
            for j in range(LANES // SC_VLEN):
                idx16 = idx_v[r, pl.ds(j * SC_VLEN, SC_VLEN)]
                for d in range(DIMS_PER_WORKER):
                    vals = plsc.load_gather(wt_v, [idx16 + (d * NUM_E)])
                    out_v[d, pl.ds(r * LANES + j * SC_VLEN, SC_VLEN)] = vals
        pltpu.sync_copy(out_v, out_hbm.at[pl.ds(d0, DIMS_PER_WORKER)])

    return _sc_gather


def kernel(z, embedding_weight):
    idx2, msum = _tc_argmin(z.T, embedding_weight.T)
    z_q = _make_sc_gather()(embedding_weight.T, idx2).T
    vq_loss = jnp.reshape(msum[0, 0] * ((1.0 + BETA) / (BATCH * DIM)), ())
    return (z_q, vq_loss)


# speedup vs baseline: 1.8598x; 1.0024x over previous
import functools

import jax
import jax.numpy as jnp
from jax import lax
from jax.experimental import pallas as pl
from jax.experimental.pallas import tpu as pltpu
from jax.experimental.pallas import tpu_sc as plsc

NUM_E = 1024
DIM = 64
BATCH = 16384
BETA = 0.25

TILE = 1024
GRID = BATCH // TILE
LANES = 128
NCHUNK = NUM_E // LANES
IDX_ROWS = BATCH // LANES

SC_CORES = 2
SC_SUBCORES = 16
SC_WORKERS = SC_CORES * SC_SUBCORES
IDXROWS_PER_WORKER = IDX_ROWS // SC_WORKERS


def _vq_argmin_kernel(zT_ref, wT_ref, idx_ref, msum_ref):
    i = pl.program_id(0)
    zT = zT_ref[...]
    wT = wT_ref[...]
    znorm = jnp.sum(zT * zT, axis=0, keepdims=True)
    wnorm = jnp.sum(wT * wT, axis=0, keepdims=True)
    wnormc = jnp.swapaxes(wnorm, 0, 1)
    tT = 2.0 * lax.dot_general(wT, zT, (((0,), (0,)), ((), ())))

    mval = None
    mchunk = None
    for k in range(NCHUNK):
        tk = lax.slice_in_dim(tT, k * LANES, (k + 1) * LANES, axis=0)
        wk = lax.slice_in_dim(wnormc, k * LANES, (k + 1) * LANES, axis=0)
        dk = (znorm + wk) - tk
        if k == 0:
            mval = dk
            mchunk = jnp.zeros(dk.shape, jnp.int32)
        else:
            lt = dk < mval
            mval = jnp.where(lt, dk, mval)
            mchunk = jnp.where(lt, k, mchunk)

    m = jnp.min(mval, axis=0, keepdims=True)
    row = lax.broadcasted_iota(jnp.int32, mval.shape, 0)
    fullidx = mchunk * LANES + row
    idx = jnp.min(jnp.where(mval == m, fullidx, NUM_E), axis=0)
    idx_ref[...] = idx.reshape(TILE // LANES, LANES)

    @pl.when(i == 0)
    def _init():
        msum_ref[0, 0] = 0.0

    msum_ref[0, 0] += jnp.sum(m)


def _tc_argmin(zT, wT):
    return pl.pallas_call(
        _vq_argmin_kernel,
        grid=(GRID,),
        in_specs=[
            pl.BlockSpec((DIM, TILE), lambda i: (0, i)),
            pl.BlockSpec((DIM, NUM_E), lambda i: (0, 0)),
        ],
        out_specs=[
            pl.BlockSpec((TILE // LANES, LANES), lambda i: (i, 0)),
            pl.BlockSpec(memory_space=pltpu.SMEM),
        ],
        out_shape=[
            jax.ShapeDtypeStruct((IDX_ROWS, LANES), jnp.int32),
            jax.ShapeDtypeStruct((1, 1), jnp.float32),
        ],
        compiler_params=pltpu.CompilerParams(
            dimension_semantics=("arbitrary",),
        ),
    )(zT, wT)


DIMS_PER_WORKER = DIM // SC_WORKERS
SC_VLEN = 16


@functools.cache
def _make_sc_gather():
    @functools.partial(
        pl.kernel,
        mesh=plsc.VectorSubcoreMesh(core_axis_name="c", subcore_axis_name="s"),
        out_type=jax.ShapeDtypeStruct((DIM, BATCH), jnp.float32),
        scratch_types=[
            pltpu.VMEM((DIMS_PER_WORKER * NUM_E,), jnp.float32),
            pltpu.VMEM((IDX_ROWS, LANES), jnp.int32),
            pltpu.VMEM((DIMS_PER_WORKER, BATCH), jnp.float32),
        ],
        compiler_params=pltpu.CompilerParams(needs_layout_passes=False),
    )
    def _sc_gather(wt_hbm, idx_hbm, out_hbm, wt_v, idx_v, out_v):
        wid = lax.axis_index("s") * SC_CORES + lax.axis_index("c")
        d0 = wid * DIMS_PER_WORKER
        for d in range(DIMS_PER_WORKER):
            pltpu.sync_copy(wt_hbm.at[d0 + d],
                            wt_v.at[pl.ds(d * NUM_E, NUM_E)])
        pltpu.sync_copy(idx_hbm, idx_v)

        @plsc.parallel_loop(0, IDX_ROWS, unroll=4)
        def _body(r):
            for j in range(LANES // SC_VLEN):
                idx16 = idx_v[r, pl.ds(j * SC_VLEN, SC_VLEN)]
                for d in range(DIMS_PER_WORKER):
                    vals = plsc.load_gather(wt_v, [idx16 + (d * NUM_E)])
                    out_v[d, pl.ds(r * LANES + j * SC_VLEN, SC_VLEN)] = vals
        pltpu.sync_copy(out_v, out_hbm.at[pl.ds(d0, DIMS_PER_WORKER)])

    return _sc_gather


def kernel(z, embedding_weight):
    idx2, msum = _tc_argmin(z.T, embedding_weight.T)
    z_q = _make_sc_gather()(embedding_weight.T, idx2).T
    vq_loss = jnp.reshape(msum[0, 0] * ((1.0 + BETA) / (BATCH * DIM)), ())
    return (z_q, vq_loss)


# speedup vs baseline: 1.9385x; 1.0423x over previous
import functools

import jax
import jax.numpy as jnp
from jax import lax
from jax.experimental import pallas as pl
from jax.experimental.pallas import tpu as pltpu
from jax.experimental.pallas import tpu_sc as plsc

NUM_E = 1024
DIM = 64
BATCH = 16384
BETA = 0.25

TILE = 2048
GRID = BATCH // TILE
LANES = 128
NCHUNK = NUM_E // LANES
IDX_ROWS = BATCH // LANES

SC_CORES = 2
SC_SUBCORES = 16
SC_WORKERS = SC_CORES * SC_SUBCORES
IDXROWS_PER_WORKER = IDX_ROWS // SC_WORKERS


def _vq_argmin_kernel(zT_ref, wT_ref, idx_ref, msum_ref):
    i = pl.program_id(0)
    zT = zT_ref[...]
    wT = wT_ref[...]
    znorm = jnp.sum(zT * zT, axis=0, keepdims=True)
    wnorm = jnp.sum(wT * wT, axis=0, keepdims=True)
    wnormc = jnp.swapaxes(wnorm, 0, 1)
    tT = 2.0 * lax.dot_general(wT, zT, (((0,), (0,)), ((), ())))

    mval = None
    mchunk = None
    for k in range(NCHUNK):
        tk = lax.slice_in_dim(tT, k * LANES, (k + 1) * LANES, axis=0)
        wk = lax.slice_in_dim(wnormc, k * LANES, (k + 1) * LANES, axis=0)
        dk = (znorm + wk) - tk
        if k == 0:
            mval = dk
            mchunk = jnp.zeros(dk.shape, jnp.int32)
        else:
            lt = dk < mval
            mval = jnp.where(lt, dk, mval)
            mchunk = jnp.where(lt, k, mchunk)

    m = jnp.min(mval, axis=0, keepdims=True)
    row = lax.broadcasted_iota(jnp.int32, mval.shape, 0)
    fullidx = mchunk * LANES + row
    idx = jnp.min(jnp.where(mval == m, fullidx, NUM_E), axis=0)
    idx_ref[...] = idx.reshape(TILE // LANES, LANES)

    @pl.when(i == 0)
    def _init():
        msum_ref[0, 0] = 0.0

    msum_ref[0, 0] += jnp.sum(m)


def _tc_argmin(zT, wT):
    return pl.pallas_call(
        _vq_argmin_kernel,
        grid=(GRID,),
        in_specs=[
            pl.BlockSpec((DIM, TILE), lambda i: (0, i)),
            pl.BlockSpec((DIM, NUM_E), lambda i: (0, 0)),
        ],
        out_specs=[
            pl.BlockSpec((TILE // LANES, LANES), lambda i: (i, 0)),
            pl.BlockSpec(memory_space=pltpu.SMEM),
        ],
        out_shape=[
            jax.ShapeDtypeStruct((IDX_ROWS, LANES), jnp.int32),
            jax.ShapeDtypeStruct((1, 1), jnp.float32),
        ],
        compiler_params=pltpu.CompilerParams(
            dimension_semantics=("arbitrary",),
        ),
    )(zT, wT)


DIMS_PER_WORKER = DIM // SC_WORKERS
SC_VLEN = 16


@functools.cache
def _make_sc_gather():
    @functools.partial(
        pl.kernel,
        mesh=plsc.VectorSubcoreMesh(core_axis_name="c", subcore_axis_name="s"),
        out_type=jax.ShapeDtypeStruct((DIM, BATCH), jnp.float32),
        scratch_types=[
            pltpu.VMEM((DIMS_PER_WORKER * NUM_E,), jnp.float32),
            pltpu.VMEM((IDX_ROWS, LANES), jnp.int32),
            pltpu.VMEM((DIMS_PER_WORKER, BATCH), jnp.float32),
        ],
        compiler_params=pltpu.CompilerParams(needs_layout_passes=False),
    )
    def _sc_gather(wt_hbm, idx_hbm, out_hbm, wt_v, idx_v, out_v):
        wid = lax.axis_index("s") * SC_CORES + lax.axis_index("c")
        d0 = wid * DIMS_PER_WORKER
        for d in range(DIMS_PER_WORKER):
            pltpu.sync_copy(wt_hbm.at[d0 + d],
                            wt_v.at[pl.ds(d * NUM_E, NUM_E)])
        pltpu.sync_copy(idx_hbm, idx_v)

        @plsc.parallel_loop(0, IDX_ROWS, unroll=4)
        def _body(r):
            for j in range(LANES // SC_VLEN):
                idx16 = idx_v[r, pl.ds(j * SC_VLEN, SC_VLEN)]
                for d in range(DIMS_PER_WORKER):
                    vals = plsc.load_gather(wt_v, [idx16 + (d * NUM_E)])
                    out_v[d, pl.ds(r * LANES + j * SC_VLEN, SC_VLEN)] = vals
        pltpu.sync_copy(out_v, out_hbm.at[pl.ds(d0, DIMS_PER_WORKER)])

    return _sc_gather


def kernel(z, embedding_weight):
    idx2, msum = _tc_argmin(z.T, embedding_weight.T)
    z_q = _make_sc_gather()(embedding_weight.T, idx2).T
    vq_loss = jnp.reshape(msum[0, 0] * ((1.0 + BETA) / (BATCH * DIM)), ())
    return (z_q, vq_loss)


# speedup vs baseline: 2.0293x; 1.0469x over previous
import functools

import jax
import jax.numpy as jnp
from jax import lax
from jax.experimental import pallas as pl
from jax.experimental.pallas import tpu as pltpu
from jax.experimental.pallas import tpu_sc as plsc

NUM_E = 1024
DIM = 64
BATCH = 16384
BETA = 0.25

TILE = 4096
GRID = BATCH // TILE
LANES = 128
NCHUNK = NUM_E // LANES
IDX_ROWS = BATCH // LANES

SC_CORES = 2
SC_SUBCORES = 16
SC_WORKERS = SC_CORES * SC_SUBCORES
IDXROWS_PER_WORKER = IDX_ROWS // SC_WORKERS


def _vq_argmin_kernel(zT_ref, wT_ref, idx_ref, msum_ref):
    i = pl.program_id(0)
    zT = zT_ref[...]
    wT = wT_ref[...]
    znorm = jnp.sum(zT * zT, axis=0, keepdims=True)
    wnorm = jnp.sum(wT * wT, axis=0, keepdims=True)
    wnormc = jnp.swapaxes(wnorm, 0, 1)
    tT = 2.0 * lax.dot_general(wT, zT, (((0,), (0,)), ((), ())))

    mval = None
    mchunk = None
    for k in range(NCHUNK):
        tk = lax.slice_in_dim(tT, k * LANES, (k + 1) * LANES, axis=0)
        wk = lax.slice_in_dim(wnormc, k * LANES, (k + 1) * LANES, axis=0)
        dk = (znorm + wk) - tk
        if k == 0:
            mval = dk
            mchunk = jnp.zeros(dk.shape, jnp.int32)
        else:
            lt = dk < mval
            mval = jnp.where(lt, dk, mval)
            mchunk = jnp.where(lt, k, mchunk)

    m = jnp.min(mval, axis=0, keepdims=True)
    row = lax.broadcasted_iota(jnp.int32, mval.shape, 0)
    fullidx = mchunk * LANES + row
    idx = jnp.min(jnp.where(mval == m, fullidx, NUM_E), axis=0)
    idx_ref[...] = idx.reshape(TILE // LANES, LANES)

    @pl.when(i == 0)
    def _init():
        msum_ref[0, 0] = 0.0

    msum_ref[0, 0] += jnp.sum(m)


def _tc_argmin(zT, wT):
    return pl.pallas_call(
        _vq_argmin_kernel,
        grid=(GRID,),
        in_specs=[
            pl.BlockSpec((DIM, TILE), lambda i: (0, i)),
            pl.BlockSpec((DIM, NUM_E), lambda i: (0, 0)),
        ],
        out_specs=[
            pl.BlockSpec((TILE // LANES, LANES), lambda i: (i, 0)),
            pl.BlockSpec(memory_space=pltpu.SMEM),
        ],
        out_shape=[
            jax.ShapeDtypeStruct((IDX_ROWS, LANES), jnp.int32),
            jax.ShapeDtypeStruct((1, 1), jnp.float32),
        ],
        compiler_params=pltpu.CompilerParams(
            dimension_semantics=("arbitrary",),
        ),
    )(zT, wT)


DIMS_PER_WORKER = DIM // SC_WORKERS
SC_VLEN = 16


@functools.cache
def _make_sc_gather():
    @functools.partial(
        pl.kernel,
        mesh=plsc.VectorSubcoreMesh(core_axis_name="c", subcore_axis_name="s"),
        out_type=jax.ShapeDtypeStruct((DIM, BATCH), jnp.float32),
        scratch_types=[
            pltpu.VMEM((DIMS_PER_WORKER * NUM_E,), jnp.float32),
            pltpu.VMEM((IDX_ROWS, LANES), jnp.int32),
            pltpu.VMEM((DIMS_PER_WORKER, BATCH), jnp.float32),
        ],
        compiler_params=pltpu.CompilerParams(needs_layout_passes=False),
    )
    def _sc_gather(wt_hbm, idx_hbm, out_hbm, wt_v, idx_v, out_v):
        wid = lax.axis_index("s") * SC_CORES + lax.axis_index("c")
        d0 = wid * DIMS_PER_WORKER
        for d in range(DIMS_PER_WORKER):
            pltpu.sync_copy(wt_hbm.at[d0 + d],
                            wt_v.at[pl.ds(d * NUM_E, NUM_E)])
        pltpu.sync_copy(idx_hbm, idx_v)

        @plsc.parallel_loop(0, IDX_ROWS, unroll=4)
        def _body(r):
            for j in range(LANES // SC_VLEN):
                idx16 = idx_v[r, pl.ds(j * SC_VLEN, SC_VLEN)]
                for d in range(DIMS_PER_WORKER):
                    vals = plsc.load_gather(wt_v, [idx16 + (d * NUM_E)])
                    out_v[d, pl.ds(r * LANES + j * SC_VLEN, SC_VLEN)] = vals
        pltpu.sync_copy(out_v, out_hbm.at[pl.ds(d0, DIMS_PER_WORKER)])

    return _sc_gather


def kernel(z, embedding_weight):
    idx2, msum = _tc_argmin(z.T, embedding_weight.T)
    z_q = _make_sc_gather()(embedding_weight.T, idx2).T
    vq_loss = jnp.reshape(msum[0, 0] * ((1.0 + BETA) / (BATCH * DIM)), ())
    return (z_q, vq_loss)


# speedup vs baseline: 2.0584x; 1.0143x over previous
import functools

import jax
import jax.numpy as jnp
from jax import lax
from jax.experimental import pallas as pl
from jax.experimental.pallas import tpu as pltpu
from jax.experimental.pallas import tpu_sc as plsc

NUM_E = 1024
DIM = 64
BATCH = 16384
BETA = 0.25

TILE = 8192
GRID = BATCH // TILE
LANES = 128
NCHUNK = NUM_E // LANES
IDX_ROWS = BATCH // LANES

SC_CORES = 2
SC_SUBCORES = 16
SC_WORKERS = SC_CORES * SC_SUBCORES
IDXROWS_PER_WORKER = IDX_ROWS // SC_WORKERS


def _vq_argmin_kernel(zT_ref, wT_ref, idx_ref, msum_ref):
    i = pl.program_id(0)
    zT = zT_ref[...]
    wT = wT_ref[...]
    znorm = jnp.sum(zT * zT, axis=0, keepdims=True)
    wnorm = jnp.sum(wT * wT, axis=0, keepdims=True)
    wnormc = jnp.swapaxes(wnorm, 0, 1)
    tT = 2.0 * lax.dot_general(wT, zT, (((0,), (0,)), ((), ())))

    mval = None
    mchunk = None
    for k in range(NCHUNK):
        tk = lax.slice_in_dim(tT, k * LANES, (k + 1) * LANES, axis=0)
        wk = lax.slice_in_dim(wnormc, k * LANES, (k + 1) * LANES, axis=0)
        dk = (znorm + wk) - tk
        if k == 0:
            mval = dk
            mchunk = jnp.zeros(dk.shape, jnp.int32)
        else:
            lt = dk < mval
            mval = jnp.where(lt, dk, mval)
            mchunk = jnp.where(lt, k, mchunk)

    m = jnp.min(mval, axis=0, keepdims=True)
    row = lax.broadcasted_iota(jnp.int32, mval.shape, 0)
    fullidx = mchunk * LANES + row
    idx = jnp.min(jnp.where(mval == m, fullidx, NUM_E), axis=0)
    idx_ref[...] = idx.reshape(TILE // LANES, LANES)

    @pl.when(i == 0)
    def _init():
        msum_ref[0, 0] = 0.0

    msum_ref[0, 0] += jnp.sum(m)


def _tc_argmin(zT, wT):
    return pl.pallas_call(
        _vq_argmin_kernel,
        grid=(GRID,),
        in_specs=[
            pl.BlockSpec((DIM, TILE), lambda i: (0, i)),
            pl.BlockSpec((DIM, NUM_E), lambda i: (0, 0)),
        ],
        out_specs=[
            pl.BlockSpec((TILE // LANES, LANES), lambda i: (i, 0)),
            pl.BlockSpec(memory_space=pltpu.SMEM),
        ],
        out_shape=[
            jax.ShapeDtypeStruct((IDX_ROWS, LANES), jnp.int32),
            jax.ShapeDtypeStruct((1, 1), jnp.float32),
        ],
        compiler_params=pltpu.CompilerParams(
            dimension_semantics=("arbitrary",),
        ),
    )(zT, wT)


DIMS_PER_WORKER = DIM // SC_WORKERS
SC_VLEN = 16


@functools.cache
def _make_sc_gather():
    @functools.partial(
        pl.kernel,
        mesh=plsc.VectorSubcoreMesh(core_axis_name="c", subcore_axis_name="s"),
        out_type=jax.ShapeDtypeStruct((DIM, BATCH), jnp.float32),
        scratch_types=[
            pltpu.VMEM((DIMS_PER_WORKER * NUM_E,), jnp.float32),
            pltpu.VMEM((IDX_ROWS, LANES), jnp.int32),
            pltpu.VMEM((DIMS_PER_WORKER, BATCH), jnp.float32),
        ],
        compiler_params=pltpu.CompilerParams(needs_layout_passes=False),
    )
    def _sc_gather(wt_hbm, idx_hbm, out_hbm, wt_v, idx_v, out_v):
        wid = lax.axis_index("s") * SC_CORES + lax.axis_index("c")
        d0 = wid * DIMS_PER_WORKER
        for d in range(DIMS_PER_WORKER):
            pltpu.sync_copy(wt_hbm.at[d0 + d],
                            wt_v.at[pl.ds(d * NUM_E, NUM_E)])
        pltpu.sync_copy(idx_hbm, idx_v)

        @plsc.parallel_loop(0, IDX_ROWS, unroll=4)
        def _body(r):
            for j in range(LANES // SC_VLEN):
                idx16 = idx_v[r, pl.ds(j * SC_VLEN, SC_VLEN)]
                for d in range(DIMS_PER_WORKER):
                    vals = plsc.load_gather(wt_v, [idx16 + (d * NUM_E)])
                    out_v[d, pl.ds(r * LANES + j * SC_VLEN, SC_VLEN)] = vals
        pltpu.sync_copy(out_v, out_hbm.at[pl.ds(d0, DIMS_PER_WORKER)])

    return _sc_gather


def kernel(z, embedding_weight):
    idx2, msum = _tc_argmin(z.T, embedding_weight.T)
    z_q = _make_sc_gather()(embedding_weight.T, idx2).T
    vq_loss = jnp.reshape(msum[0, 0] * ((1.0 + BETA) / (BATCH * DIM)), ())
    return (z_q, vq_loss)
